# baseline (speedup 1.0000x reference)
import functools

import jax
import jax.numpy as jnp
from jax import lax
from jax.experimental import pallas as pl
from jax.experimental.pallas import tpu as pltpu
from jax.experimental.pallas import tpu_sc as plsc

B, L, F, D = 4096, 50, 5, 64
N = B * L

NC, NS = 2, 16
NW = NC * NS
CH = 256
CPF = B * L // CH // NW
IPF = CPF * CH


def _emb(idxT, t0, t1, t2, t3, t4):
    mesh = plsc.VectorSubcoreMesh(core_axis_name="c", subcore_axis_name="s")

    @functools.partial(
        pl.kernel,
        out_type=jax.ShapeDtypeStruct((L, F, D, B), jnp.float32),
        mesh=mesh,
        scratch_types=[
            pltpu.VMEM((F * IPF,), jnp.int32),
            pltpu.VMEM((CH, D), jnp.float32),
            pltpu.VMEM((D, CH), jnp.float32),
            pltpu.SemaphoreType.DMA,
            pltpu.SemaphoreType.DMA,
        ],
        compiler_params=pltpu.CompilerParams(use_tc_tiling_on_sc=False,
                                             needs_layout_passes=False),
    )
    def body(idx_hbm, T0, T1, T2, T3, T4, out_hbm,
             idx_all, rows_v, trans_v, gsem, ssem):
        tables = [T0, T1, T2, T3, T4]
        wid = lax.axis_index("s") * NC + lax.axis_index("c")
        wbase = pl.multiple_of(wid * IPF, 8)

        for f in range(F):
            pltpu.sync_copy(idx_hbm.at[pl.ds(f * N + wbase, IPF)],
                            idx_all.at[pl.ds(f * IPF, IPF)])

        iota16 = lax.iota(jnp.int32, 16)

        for f in range(F):
            tab = tables[f]

            def chunk(u, _, tab=tab, f=f):
                g = wid * CPF + u
                l = g // (B // CH)
                b0 = pl.multiple_of((g % (B // CH)) * CH, 8)
                idx = idx_all.at[pl.ds(f * IPF + u * CH, CH)]
                pltpu.async_copy(tab.at[idx], rows_v, gsem).wait()

                def drow(d, _):
                    col = jnp.full((16,), 0, jnp.int32) + d
                    for j in range(CH // 16):
                        row = iota16 + (j * 16)
                        v = plsc.load_gather(rows_v, [row, col])
                        trans_v[d, pl.ds(j * 16, 16)] = v
                    return ()

                lax.fori_loop(0, D, drow, ())
                pltpu.async_copy(
                    trans_v, out_hbm.at[l, f, :, pl.ds(b0, CH)], ssem).wait()
                return ()

            lax.fori_loop(0, CPF, chunk, ())

    return body(idxT, t0, t1, t2, t3, t4)


def kernel(input, T0, T1, T2, T3, T4):
    idxT = jnp.transpose(input, (2, 1, 0)).reshape(-1)
    out = _emb(idxT, T0, T1, T2, T3, T4)
    return jnp.transpose(out, (3, 0, 1, 2))


# speedup vs baseline: 1.1214x; 1.1214x over previous
import functools

import jax
import jax.numpy as jnp
from jax import lax
from jax.experimental import pallas as pl
from jax.experimental.pallas import tpu as pltpu
from jax.experimental.pallas import tpu_sc as plsc

B, L, F, D = 4096, 50, 5, 64
N = B * L

NC, NS = 2, 16
NW = NC * NS
CH = 128
NBC = B // CH
CPF = L * NBC // NW
IPF = CPF * CH


def _emb(idxT, t0, t1, t2, t3, t4):
    mesh = plsc.VectorSubcoreMesh(core_axis_name="c", subcore_axis_name="s")

    @functools.partial(
        pl.kernel,
        out_type=jax.ShapeDtypeStruct((L, F, D, B), jnp.float32),
        mesh=mesh,
        scratch_types=[
            pltpu.VMEM((F * IPF,), jnp.int32),
            pltpu.VMEM((CH, D), jnp.float32),
            pltpu.VMEM((CH, D), jnp.float32),
            pltpu.VMEM((D, CH), jnp.float32),
            pltpu.VMEM((D, CH), jnp.float32),
            pltpu.SemaphoreType.DMA,
            pltpu.SemaphoreType.DMA,
            pltpu.SemaphoreType.DMA,
            pltpu.SemaphoreType.DMA,
        ],
        compiler_params=pltpu.CompilerParams(use_tc_tiling_on_sc=False,
                                             needs_layout_passes=False),
    )
    def body(idx_hbm, T0, T1, T2, T3, T4, out_hbm,
             idx_all, rows0, rows1, tr0, tr1, gs0, gs1, ss0, ss1):
        tables = [T0, T1, T2, T3, T4]
        rows, trs = [rows0, rows1], [tr0, tr1]
        gsem, ssem = [gs0, gs1], [ss0, ss1]
        wid = lax.axis_index("s") * NC + lax.axis_index("c")
        wbase = pl.multiple_of(wid * IPF, 8)

        for f in range(F):
            pltpu.sync_copy(idx_hbm.at[pl.ds(f * N + wbase, IPF)],
                            idx_all.at[pl.ds(f * IPF, IPF)])

        iota16 = lax.iota(jnp.int32, 16)
        zero16 = iota16 * 0

        def start_gather(f, c, b):
            idx = idx_all.at[pl.ds(f * IPF + c * CH, CH)]
            pltpu.async_copy(tables[f].at[idx], rows[b], gsem[b])

        def wait_gather(b):
            pltpu.make_async_copy(
                tables[0].at[pl.ds(0, CH)], rows[b], gsem[b]).wait()

        def start_out(f, c, b):
            g = wid * CPF + c
            l = g // NBC
            b0 = pl.multiple_of((g % NBC) * CH, 8)
            pltpu.async_copy(
                trs[b], out_hbm.at[l, f, :, pl.ds(b0, CH)], ssem[b])

        def wait_out(b):
            pltpu.make_async_copy(
                trs[b], out_hbm.at[0, 0, :, pl.ds(0, CH)], ssem[b]).wait()

        def transpose(b):
            src, dst = rows[b], trs[b]

            def dloop(d, col):
                for j in range(CH // 16):
                    row = iota16 + j * 16
                    v = plsc.load_gather(src, [row, col])
                    dst[d, pl.ds(j * 16, 16)] = v
                return col + 1

            lax.fori_loop(0, D, dloop, zero16)

        for f in range(F):
            start_gather(f, 0, 0)
            start_gather(f, 1, 1)
            wait_gather(0)
            transpose(0)
            start_out(f, 0, 0)
            start_gather(f, 2, 0)
            wait_gather(1)
            transpose(1)
            start_out(f, 1, 1)
            start_gather(f, 3, 1)

            def step(p, _, f=f):
                c0 = 2 * p
                wait_gather(0)
                wait_out(0)
                transpose(0)
                start_out(f, c0, 0)
                start_gather(f, c0 + 2, 0)
                wait_gather(1)
                wait_out(1)
                transpose(1)
                start_out(f, c0 + 1, 1)
                start_gather(f, c0 + 3, 1)
                return ()

            lax.fori_loop(1, 24, step, ())

            for c in (48, 49):
                b = c % 2
                wait_gather(b)
                wait_out(b)
                transpose(b)
                start_out(f, c, b)
            wait_out(0)
            wait_out(1)

    return body(idxT, t0, t1, t2, t3, t4)


def kernel(input, T0, T1, T2, T3, T4):
    idxT = jnp.transpose(input, (2, 1, 0)).reshape(-1)
    out = _emb(idxT, T0, T1, T2, T3, T4)
    return jnp.transpose(out, (3, 0, 1, 2))


# speedup vs baseline: 1.8939x; 1.6889x over previous
import functools

import jax
import jax.numpy as jnp
from jax import lax
from jax.experimental import pallas as pl
from jax.experimental.pallas import tpu as pltpu
from jax.experimental.pallas import tpu_sc as plsc

B, L, F, D = 4096, 50, 5, 64
N = B * L

NC, NS = 2, 16
NW = NC * NS
RPW = N // NW
CH = 640
NB = RPW // CH


def _emb(idxT, t0, t1, t2, t3, t4):
    mesh = plsc.VectorSubcoreMesh(core_axis_name="c", subcore_axis_name="s")

    @functools.partial(
        pl.kernel,
        out_type=jax.ShapeDtypeStruct((N, F, D), jnp.float32),
        mesh=mesh,
        scratch_types=[
            pltpu.VMEM((F * RPW,), jnp.int32),
            pltpu.VMEM((CH, 1, D), jnp.float32),
            pltpu.VMEM((CH, 1, D), jnp.float32),
            pltpu.SemaphoreType.DMA,
            pltpu.SemaphoreType.DMA,
            pltpu.SemaphoreType.DMA,
            pltpu.SemaphoreType.DMA,
        ],
        compiler_params=pltpu.CompilerParams(use_tc_tiling_on_sc=False),
    )
    def body(idx_hbm, T0, T1, T2, T3, T4, out_hbm,
             idx_all, rows0, rows1, gs0, gs1, ss0, ss1):
        tables = [T0, T1, T2, T3, T4]
        bufs, gsem, ssem = [rows0, rows1], [gs0, gs1], [ss0, ss1]
        wid = lax.axis_index("s") * NC + lax.axis_index("c")
        wbase = pl.multiple_of(wid * RPW, 8)

        for f in range(F):
            pltpu.sync_copy(idx_hbm.at[pl.ds(f * N + wbase, RPW)],
                            idx_all.at[pl.ds(f * RPW, RPW)])

        T = F * NB
        gath, scat = [None, None], [None, None]

        def start_gather(t):
            f, i, b = t // NB, t % NB, t % 2
            idx = idx_all.at[pl.ds((f * NB + i) * CH, CH)]
            gath[b] = pltpu.async_copy(tables[f].at[idx], bufs[b].at[:, 0],
                                       gsem[b])

        def start_scatter(t):
            f, i, b = t // NB, t % NB, t % 2
            n0 = pl.multiple_of(wbase + i * CH, 8)
            scat[b] = pltpu.async_copy(
                bufs[b], out_hbm.at[pl.ds(n0, CH), pl.ds(f, 1)], ssem[b])

        start_gather(0)
        for t in range(T):
            b, nb = t % 2, (t + 1) % 2
            if t + 1 < T:
                if scat[nb] is not None:
                    scat[nb].wait()
                start_gather(t + 1)
            gath[b].wait()
            start_scatter(t)
        scat[0].wait()
        scat[1].wait()

    return body(idxT, t0, t1, t2, t3, t4)


def kernel(input, T0, T1, T2, T3, T4):
    idxT = jnp.transpose(input, (2, 1, 0)).reshape(-1)
    out = _emb(idxT, T0, T1, T2, T3, T4)
    return jnp.transpose(out.reshape(L, B, F, D), (1, 0, 2, 3))
